# baseline (speedup 1.0000x reference)
import math

import jax
import jax.numpy as jnp
from jax import lax
from jax.experimental import pallas as pl
from jax.experimental.pallas import tpu as pltpu
from jax.experimental.pallas import tpu_sc as plsc

_N = 2_000_000
_NSAMP = 100_000
_SMOOTH = 0.9
_LN_SMOOTH = math.log(_SMOOTH)

_NC = 2
_NS = 16

_BINS_PER_TILE = 6256
_BINS_PAD = _NS * _BINS_PER_TILE

_P1_CHUNK = 5000
_P1_HALF1 = 2504
_P1_HALF2 = _P1_CHUNK - _P1_HALF1
_P1_NCHUNKS = _N // _P1_CHUNK

_P3_PER_TILE = 62560
_P3_CHUNK = 6256
_P3_HALF = _P3_CHUNK // 2
_P3_ITERS = _P3_PER_TILE // _P3_CHUNK


def _acc_body(values_hbm, indices_hbm, acc_hbm,
              sums_s, cnts_s,
              vals_a, vals_b, idx_a1, idx_a2, idx_b1, idx_b2,
              ones_v, zero_v,
              sva, sia, svb, sib, ssv, ssc):
  cid = lax.axis_index("c")
  sid = lax.axis_index("s")

  def _fill(i, _):
    zero_v[pl.ds(i * 16, 16)] = jnp.zeros((16,), jnp.float32)
    return 0
  lax.fori_loop(0, _BINS_PER_TILE // 16, _fill, 0)

  def _fill1(i, _):
    ones_v[pl.ds(i * 16, 16)] = jnp.ones((16,), jnp.float32)
    return 0
  lax.fori_loop(0, ones_v.shape[0] // 16, _fill1, 0)

  b0 = pl.multiple_of(sid * _BINS_PER_TILE, 8)
  pltpu.sync_copy(zero_v, sums_s.at[pl.ds(b0, _BINS_PER_TILE)])
  pltpu.sync_copy(zero_v, cnts_s.at[pl.ds(b0, _BINS_PER_TILE)])
  plsc.subcore_barrier()

  wid = sid * _NC + cid

  def _load(j, vbuf, ibuf1, ibuf2, vsem, isem):
    base = pl.multiple_of((wid + 32 * j) * _P1_CHUNK, 8)
    pltpu.async_copy(values_hbm.at[pl.ds(base, _P1_CHUNK)], vbuf, vsem)
    pltpu.async_copy(indices_hbm.at[pl.ds(base, _P1_HALF1)], ibuf1, isem)
    pltpu.async_copy(
        indices_hbm.at[pl.ds(pl.multiple_of(base + _P1_HALF1, 8),
                             _P1_HALF2)], ibuf2, isem)

  def _wait_load(vbuf, ibuf1, ibuf2, vsem, isem):
    pltpu.make_async_copy(values_hbm.at[pl.ds(0, _P1_CHUNK)], vbuf,
                          vsem).wait()
    pltpu.make_async_copy(indices_hbm.at[pl.ds(0, _P1_HALF1)], ibuf1,
                          isem).wait()
    pltpu.make_async_copy(indices_hbm.at[pl.ds(0, _P1_HALF2)], ibuf2,
                          isem).wait()

  def _scatter(vbuf, ibuf1, ibuf2):
    d1 = pltpu.async_copy(vbuf.at[pl.ds(0, _P1_HALF1)],
                          sums_s.at[ibuf1], ssv, add=True)
    d2 = pltpu.async_copy(vbuf.at[pl.ds(_P1_HALF1, _P1_HALF2)],
                          sums_s.at[ibuf2], ssv, add=True)
    d3 = pltpu.async_copy(ones_v.at[pl.ds(0, _P1_HALF1)],
                          cnts_s.at[ibuf1], ssc, add=True)
    d4 = pltpu.async_copy(ones_v.at[pl.ds(0, _P1_HALF2)],
                          cnts_s.at[ibuf2], ssc, add=True)
    d1.wait()
    d2.wait()
    d3.wait()
    d4.wait()

  _load(0, vals_a, idx_a1, idx_a2, sva, sia)

  def _pipe(g, _):
    _wait_load(vals_a, idx_a1, idx_a2, sva, sia)
    _load(2 * g + 1, vals_b, idx_b1, idx_b2, svb, sib)
    _scatter(vals_a, idx_a1, idx_a2)
    _wait_load(vals_b, idx_b1, idx_b2, svb, sib)

    @pl.when(g < 5)
    def _pf():
      _load(2 * g + 2, vals_a, idx_a1, idx_a2, sva, sia)

    _scatter(vals_b, idx_b1, idx_b2)
    return 0
  lax.fori_loop(0, 6, _pipe, 0)

  @pl.when(wid < 16)
  def _extra():
    _load(12, vals_a, idx_a1, idx_a2, sva, sia)
    _wait_load(vals_a, idx_a1, idx_a2, sva, sia)
    _scatter(vals_a, idx_a1, idx_a2)

  plsc.subcore_barrier()

  po = pl.multiple_of(cid * 2 * _BINS_PAD + b0, 8)
  pltpu.sync_copy(sums_s.at[pl.ds(b0, _BINS_PER_TILE)], zero_v)
  pltpu.sync_copy(zero_v, acc_hbm.at[pl.ds(po, _BINS_PER_TILE)])
  pltpu.sync_copy(cnts_s.at[pl.ds(b0, _BINS_PER_TILE)], zero_v)
  pltpu.sync_copy(zero_v, acc_hbm.at[pl.ds(po + _BINS_PAD, _BINS_PER_TILE)])


def _ema_gather_body(acc_hbm, indices_hbm, memory_hbm, out_hbm,
                     newm_s, s0_v, s1_v, c0_v, c1_v, mem_v, newm_v,
                     idx_a1, idx_a2, idx_b1, idx_b2, out_a, out_b,
                     sia, sib, soa, sob, sg):
  cid = lax.axis_index("c")
  sid = lax.axis_index("s")
  wid = sid * _NC + cid

  b0 = pl.multiple_of(sid * _BINS_PER_TILE, 8)
  pltpu.sync_copy(acc_hbm.at[pl.ds(b0, _BINS_PER_TILE)], s0_v)
  pltpu.sync_copy(acc_hbm.at[pl.ds(2 * _BINS_PAD + b0, _BINS_PER_TILE)], s1_v)
  pltpu.sync_copy(acc_hbm.at[pl.ds(_BINS_PAD + b0, _BINS_PER_TILE)], c0_v)
  pltpu.sync_copy(acc_hbm.at[pl.ds(3 * _BINS_PAD + b0, _BINS_PER_TILE)], c1_v)
  pltpu.sync_copy(memory_hbm.at[pl.ds(b0, _BINS_PER_TILE)], mem_v)

  def _p2(k, _):
    o = k * 16
    s = s0_v[pl.ds(o, 16)] + s1_v[pl.ds(o, 16)]
    c = c0_v[pl.ds(o, 16)] + c1_v[pl.ds(o, 16)]
    m = mem_v[pl.ds(o, 16)]
    mean = s / jnp.maximum(c, 1.0)
    alpha = jnp.exp(c * _LN_SMOOTH)
    nm = jnp.where(c > 0.0, alpha * m + (1.0 - alpha) * mean, m)
    newm_v[pl.ds(o, 16)] = nm
    return 0
  lax.fori_loop(0, _BINS_PER_TILE // 16, _p2, 0)

  pltpu.sync_copy(newm_v, newm_s.at[pl.ds(b0, _BINS_PER_TILE)])
  plsc.subcore_barrier()

  p3_base = jnp.minimum(wid * _P3_PER_TILE, _N - _P3_PER_TILE)

  def _cbase(j):
    return pl.multiple_of(p3_base + j * _P3_CHUNK, 8)

  def _load_idx(j, ibuf1, ibuf2, isem):
    pltpu.async_copy(indices_hbm.at[pl.ds(_cbase(j), _P3_HALF)], ibuf1,
                     isem)
    pltpu.async_copy(
        indices_hbm.at[pl.ds(pl.multiple_of(_cbase(j) + _P3_HALF, 8),
                             _P3_HALF)], ibuf2, isem)

  def _wait_idx(ibuf1, ibuf2, isem):
    pltpu.make_async_copy(indices_hbm.at[pl.ds(0, _P3_HALF)], ibuf1,
                          isem).wait()
    pltpu.make_async_copy(indices_hbm.at[pl.ds(0, _P3_HALF)], ibuf2,
                          isem).wait()

  def _gather(ibuf1, ibuf2, obuf):
    d1 = pltpu.async_copy(newm_s.at[ibuf1], obuf.at[pl.ds(0, _P3_HALF)],
                          sg)
    d2 = pltpu.async_copy(newm_s.at[ibuf2],
                          obuf.at[pl.ds(_P3_HALF, _P3_HALF)], sg)
    d1.wait()
    d2.wait()

  def _drain_out(obuf, osem):
    pltpu.make_async_copy(obuf, out_hbm.at[pl.ds(0, _P3_CHUNK)],
                          osem).wait()

  _load_idx(0, idx_a1, idx_a2, sia)

  def _pipe(g, _):
    _wait_idx(idx_a1, idx_a2, sia)
    _load_idx(2 * g + 1, idx_b1, idx_b2, sib)

    @pl.when(g > 0)
    def _da():
      _drain_out(out_a, soa)

    _gather(idx_a1, idx_a2, out_a)
    pltpu.async_copy(out_a, out_hbm.at[pl.ds(_cbase(2 * g), _P3_CHUNK)],
                     soa)
    _wait_idx(idx_b1, idx_b2, sib)

    @pl.when(g < _P3_ITERS // 2 - 1)
    def _pf():
      _load_idx(2 * g + 2, idx_a1, idx_a2, sia)

    @pl.when(g > 0)
    def _db():
      _drain_out(out_b, sob)

    _gather(idx_b1, idx_b2, out_b)
    pltpu.async_copy(out_b,
                     out_hbm.at[pl.ds(_cbase(2 * g + 1), _P3_CHUNK)], sob)
    return 0
  lax.fori_loop(0, _P3_ITERS // 2, _pipe, 0)

  _drain_out(out_a, soa)
  _drain_out(out_b, sob)


@jax.jit
def _smoother(values, indices, memory_padded):
  mesh = plsc.VectorSubcoreMesh(core_axis_name="c", subcore_axis_name="s")
  acc = pl.kernel(
      _acc_body,
      out_type=jax.ShapeDtypeStruct((_NC * 2 * _BINS_PAD,), jnp.float32),
      mesh=mesh,
      scratch_types=[
          pltpu.VMEM_SHARED((_BINS_PAD,), jnp.float32),
          pltpu.VMEM_SHARED((_BINS_PAD,), jnp.float32),
          pltpu.VMEM((_P1_CHUNK,), jnp.float32),
          pltpu.VMEM((_P1_CHUNK,), jnp.float32),
          pltpu.VMEM((_P1_HALF1,), jnp.int32),
          pltpu.VMEM((_P1_HALF2,), jnp.int32),
          pltpu.VMEM((_P1_HALF1,), jnp.int32),
          pltpu.VMEM((_P1_HALF2,), jnp.int32),
          pltpu.VMEM((5008,), jnp.float32),
          pltpu.VMEM((_BINS_PER_TILE,), jnp.float32),
          pltpu.SemaphoreType.DMA,
          pltpu.SemaphoreType.DMA,
          pltpu.SemaphoreType.DMA,
          pltpu.SemaphoreType.DMA,
          pltpu.SemaphoreType.DMA,
          pltpu.SemaphoreType.DMA,
      ],
  )(values, indices)

  return pl.kernel(
      _ema_gather_body,
      out_type=jax.ShapeDtypeStruct((_N,), jnp.float32),
      mesh=mesh,
      scratch_types=[
          pltpu.VMEM_SHARED((_BINS_PAD,), jnp.float32),
          pltpu.VMEM((_BINS_PER_TILE,), jnp.float32),
          pltpu.VMEM((_BINS_PER_TILE,), jnp.float32),
          pltpu.VMEM((_BINS_PER_TILE,), jnp.float32),
          pltpu.VMEM((_BINS_PER_TILE,), jnp.float32),
          pltpu.VMEM((_BINS_PER_TILE,), jnp.float32),
          pltpu.VMEM((_BINS_PER_TILE,), jnp.float32),
          pltpu.VMEM((_P3_HALF,), jnp.int32),
          pltpu.VMEM((_P3_HALF,), jnp.int32),
          pltpu.VMEM((_P3_HALF,), jnp.int32),
          pltpu.VMEM((_P3_HALF,), jnp.int32),
          pltpu.VMEM((_P3_CHUNK,), jnp.float32),
          pltpu.VMEM((_P3_CHUNK,), jnp.float32),
          pltpu.SemaphoreType.DMA,
          pltpu.SemaphoreType.DMA,
          pltpu.SemaphoreType.DMA,
          pltpu.SemaphoreType.DMA,
          pltpu.SemaphoreType.DMA,
      ],
  )(acc, indices, memory_padded)


def kernel(values, indices, memory):
  memory_padded = jnp.concatenate(
      [memory, jnp.zeros((_BINS_PAD - _NSAMP,), jnp.float32)])
  return _smoother(values, indices, memory_padded)


# speedup vs baseline: 1.2164x; 1.2164x over previous
import math

import jax
import jax.numpy as jnp
from jax import lax
from jax.experimental import pallas as pl
from jax.experimental.pallas import tpu as pltpu
from jax.experimental.pallas import tpu_sc as plsc

_N = 2_000_000
_NSAMP = 100_000
_SMOOTH = 0.9
_LN_SMOOTH = math.log(_SMOOTH)

_NC = 2
_NS = 16

_BINS_PER_TILE = 6256
_BINS_PAD = _NS * _BINS_PER_TILE

_P1_CHUNK = 5000
_P1_NCHUNKS = _N // _P1_CHUNK

_P3_PER_TILE = 62560
_P3_CHUNK = 6256
_P3_ITERS = _P3_PER_TILE // _P3_CHUNK


def _acc_body(values_hbm, indices_hbm, acc_hbm,
              sums_s, cnts_s,
              vals_a, vals_b, idx_a, idx_b, ones_v, zero_v,
              sva, sia, svb, sib, ssv, ssc):
  cid = lax.axis_index("c")
  sid = lax.axis_index("s")
  wid = sid * _NC + cid

  def _load(j, vbuf, ibuf, vsem, isem):
    base = pl.multiple_of((wid + 32 * j) * _P1_CHUNK, 8)
    pltpu.async_copy(values_hbm.at[pl.ds(base, _P1_CHUNK)], vbuf, vsem)
    pltpu.async_copy(indices_hbm.at[pl.ds(base, _P1_CHUNK)], ibuf, isem)

  def _wait_load(vbuf, ibuf, vsem, isem):
    pltpu.make_async_copy(values_hbm.at[pl.ds(0, _P1_CHUNK)], vbuf,
                          vsem).wait()
    pltpu.make_async_copy(indices_hbm.at[pl.ds(0, _P1_CHUNK)], ibuf,
                          isem).wait()

  def _scatter(vbuf, ibuf):
    d1 = pltpu.async_copy(vbuf, sums_s.at[ibuf], ssv, add=True)
    d2 = pltpu.async_copy(ones_v.at[pl.ds(0, _P1_CHUNK)],
                          cnts_s.at[ibuf], ssc, add=True)
    d1.wait()
    d2.wait()

  _load(0, vals_a, idx_a, sva, sia)
  _load(1, vals_b, idx_b, svb, sib)

  def _fill(i, _):
    zero_v[pl.ds(i * 16, 16)] = jnp.zeros((16,), jnp.float32)
    return 0
  lax.fori_loop(0, _BINS_PER_TILE // 16, _fill, 0)

  def _fill1(i, _):
    ones_v[pl.ds(i * 16, 16)] = jnp.ones((16,), jnp.float32)
    return 0
  lax.fori_loop(0, ones_v.shape[0] // 16, _fill1, 0)

  b0 = pl.multiple_of(sid * _BINS_PER_TILE, 8)
  pltpu.sync_copy(zero_v, sums_s.at[pl.ds(b0, _BINS_PER_TILE)])
  pltpu.sync_copy(zero_v, cnts_s.at[pl.ds(b0, _BINS_PER_TILE)])
  plsc.subcore_barrier()

  def _pipe(g, _):
    _wait_load(vals_a, idx_a, sva, sia)
    _scatter(vals_a, idx_a)

    @pl.when(g < 5)
    def _pfa():
      _load(2 * g + 2, vals_a, idx_a, sva, sia)

    _wait_load(vals_b, idx_b, svb, sib)
    _scatter(vals_b, idx_b)

    @pl.when(g < 5)
    def _pfb():
      _load(2 * g + 3, vals_b, idx_b, svb, sib)

    return 0
  lax.fori_loop(0, 6, _pipe, 0)

  @pl.when(wid < 16)
  def _extra():
    _load(12, vals_a, idx_a, sva, sia)
    _wait_load(vals_a, idx_a, sva, sia)
    _scatter(vals_a, idx_a)

  plsc.subcore_barrier()

  po = pl.multiple_of(cid * 2 * _BINS_PAD + b0, 8)
  pltpu.sync_copy(sums_s.at[pl.ds(b0, _BINS_PER_TILE)], zero_v)
  pltpu.sync_copy(zero_v, acc_hbm.at[pl.ds(po, _BINS_PER_TILE)])
  pltpu.sync_copy(cnts_s.at[pl.ds(b0, _BINS_PER_TILE)], zero_v)
  pltpu.sync_copy(zero_v, acc_hbm.at[pl.ds(po + _BINS_PAD, _BINS_PER_TILE)])


def _ema_gather_body(acc_hbm, indices_hbm, memory_hbm, out_hbm,
                     newm_s, s0_v, s1_v, c0_v, c1_v, mem_v, newm_v,
                     idx_a, idx_b, out_a, out_b,
                     sia, sib, soa, sob, sld):
  cid = lax.axis_index("c")
  sid = lax.axis_index("s")
  wid = sid * _NC + cid

  b0 = pl.multiple_of(sid * _BINS_PER_TILE, 8)
  d1 = pltpu.async_copy(acc_hbm.at[pl.ds(b0, _BINS_PER_TILE)], s0_v, sld)
  d2 = pltpu.async_copy(
      acc_hbm.at[pl.ds(2 * _BINS_PAD + b0, _BINS_PER_TILE)], s1_v, sld)
  d3 = pltpu.async_copy(
      acc_hbm.at[pl.ds(_BINS_PAD + b0, _BINS_PER_TILE)], c0_v, sld)
  d4 = pltpu.async_copy(
      acc_hbm.at[pl.ds(3 * _BINS_PAD + b0, _BINS_PER_TILE)], c1_v, sld)
  d5 = pltpu.async_copy(memory_hbm.at[pl.ds(b0, _BINS_PER_TILE)], mem_v,
                        sld)
  d1.wait()
  d2.wait()
  d3.wait()
  d4.wait()
  d5.wait()

  def _p2(k, _):
    o = k * 16
    s = s0_v[pl.ds(o, 16)] + s1_v[pl.ds(o, 16)]
    c = c0_v[pl.ds(o, 16)] + c1_v[pl.ds(o, 16)]
    m = mem_v[pl.ds(o, 16)]
    mean = s / jnp.maximum(c, 1.0)
    alpha = jnp.exp(c * _LN_SMOOTH)
    nm = jnp.where(c > 0.0, alpha * m + (1.0 - alpha) * mean, m)
    newm_v[pl.ds(o, 16)] = nm
    return 0
  lax.fori_loop(0, _BINS_PER_TILE // 16, _p2, 0)

  pltpu.sync_copy(newm_v, newm_s.at[pl.ds(b0, _BINS_PER_TILE)])
  plsc.subcore_barrier()

  p3_base = jnp.minimum(wid * _P3_PER_TILE, _N - _P3_PER_TILE)

  def _cbase(j):
    return pl.multiple_of(p3_base + j * _P3_CHUNK, 8)

  def _load_idx(j, ibuf, isem):
    pltpu.async_copy(indices_hbm.at[pl.ds(_cbase(j), _P3_CHUNK)], ibuf,
                     isem)

  def _wait_idx(ibuf, isem):
    pltpu.make_async_copy(indices_hbm.at[pl.ds(0, _P3_CHUNK)], ibuf,
                          isem).wait()

  def _drain_out(obuf, osem):
    pltpu.make_async_copy(obuf, out_hbm.at[pl.ds(0, _P3_CHUNK)],
                          osem).wait()

  _load_idx(0, idx_a, sia)

  def _pipe(g, _):
    _wait_idx(idx_a, sia)
    _load_idx(2 * g + 1, idx_b, sib)

    @pl.when(g > 0)
    def _da():
      _drain_out(out_a, soa)

    pltpu.sync_copy(newm_s.at[idx_a], out_a)
    pltpu.async_copy(out_a, out_hbm.at[pl.ds(_cbase(2 * g), _P3_CHUNK)],
                     soa)
    _wait_idx(idx_b, sib)

    @pl.when(g < _P3_ITERS // 2 - 1)
    def _pf():
      _load_idx(2 * g + 2, idx_a, sia)

    @pl.when(g > 0)
    def _db():
      _drain_out(out_b, sob)

    pltpu.sync_copy(newm_s.at[idx_b], out_b)
    pltpu.async_copy(out_b,
                     out_hbm.at[pl.ds(_cbase(2 * g + 1), _P3_CHUNK)], sob)
    return 0
  lax.fori_loop(0, _P3_ITERS // 2, _pipe, 0)

  _drain_out(out_a, soa)
  _drain_out(out_b, sob)


@jax.jit
def _smoother(values, indices, memory_padded):
  mesh = plsc.VectorSubcoreMesh(core_axis_name="c", subcore_axis_name="s")
  acc = pl.kernel(
      _acc_body,
      out_type=jax.ShapeDtypeStruct((_NC * 2 * _BINS_PAD,), jnp.float32),
      mesh=mesh,
      scratch_types=[
          pltpu.VMEM_SHARED((_BINS_PAD,), jnp.float32),
          pltpu.VMEM_SHARED((_BINS_PAD,), jnp.float32),
          pltpu.VMEM((_P1_CHUNK,), jnp.float32),
          pltpu.VMEM((_P1_CHUNK,), jnp.float32),
          pltpu.VMEM((_P1_CHUNK,), jnp.int32),
          pltpu.VMEM((_P1_CHUNK,), jnp.int32),
          pltpu.VMEM((5008,), jnp.float32),
          pltpu.VMEM((_BINS_PER_TILE,), jnp.float32),
          pltpu.SemaphoreType.DMA,
          pltpu.SemaphoreType.DMA,
          pltpu.SemaphoreType.DMA,
          pltpu.SemaphoreType.DMA,
          pltpu.SemaphoreType.DMA,
          pltpu.SemaphoreType.DMA,
      ],
  )(values, indices)

  return pl.kernel(
      _ema_gather_body,
      out_type=jax.ShapeDtypeStruct((_N,), jnp.float32),
      mesh=mesh,
      scratch_types=[
          pltpu.VMEM_SHARED((_BINS_PAD,), jnp.float32),
          pltpu.VMEM((_BINS_PER_TILE,), jnp.float32),
          pltpu.VMEM((_BINS_PER_TILE,), jnp.float32),
          pltpu.VMEM((_BINS_PER_TILE,), jnp.float32),
          pltpu.VMEM((_BINS_PER_TILE,), jnp.float32),
          pltpu.VMEM((_BINS_PER_TILE,), jnp.float32),
          pltpu.VMEM((_BINS_PER_TILE,), jnp.float32),
          pltpu.VMEM((_P3_CHUNK,), jnp.int32),
          pltpu.VMEM((_P3_CHUNK,), jnp.int32),
          pltpu.VMEM((_P3_CHUNK,), jnp.float32),
          pltpu.VMEM((_P3_CHUNK,), jnp.float32),
          pltpu.SemaphoreType.DMA,
          pltpu.SemaphoreType.DMA,
          pltpu.SemaphoreType.DMA,
          pltpu.SemaphoreType.DMA,
          pltpu.SemaphoreType.DMA,
      ],
  )(acc, indices, memory_padded)


def kernel(values, indices, memory):
  memory_padded = jnp.concatenate(
      [memory, jnp.zeros((_BINS_PAD - _NSAMP,), jnp.float32)])
  return _smoother(values, indices, memory_padded)


# speedup vs baseline: 1.2251x; 1.0071x over previous
import math

import jax
import jax.numpy as jnp
from jax import lax
from jax.experimental import pallas as pl
from jax.experimental.pallas import tpu as pltpu
from jax.experimental.pallas import tpu_sc as plsc

_N = 2_000_000
_NSAMP = 100_000
_SMOOTH = 0.9
_LN_SMOOTH = math.log(_SMOOTH)

_NC = 2
_NS = 16

_BINS_PER_TILE = 6256
_BINS_PAD = _NS * _BINS_PER_TILE

_P1_CHUNK = 5000
_P1_NCHUNKS = _N // _P1_CHUNK

_P3_PER_TILE = 62560
_P3_CHUNK = 6256
_P3_ITERS = _P3_PER_TILE // _P3_CHUNK


def _acc_body(values_hbm, indices_hbm, acc_hbm,
              sums_s, cnts_s,
              vals_a, vals_b, idx_a, idx_b, ones_v, zero_v, stg_v,
              sva, sia, svb, sib, ssva, ssca, ssvb, sscb):
  cid = lax.axis_index("c")
  sid = lax.axis_index("s")
  wid = sid * _NC + cid

  def _load(j, vbuf, ibuf, vsem, isem):
    base = pl.multiple_of((wid + 32 * j) * _P1_CHUNK, 8)
    pltpu.async_copy(values_hbm.at[pl.ds(base, _P1_CHUNK)], vbuf, vsem)
    pltpu.async_copy(indices_hbm.at[pl.ds(base, _P1_CHUNK)], ibuf, isem)

  def _wait_load(vbuf, ibuf, vsem, isem):
    pltpu.make_async_copy(values_hbm.at[pl.ds(0, _P1_CHUNK)], vbuf,
                          vsem).wait()
    pltpu.make_async_copy(indices_hbm.at[pl.ds(0, _P1_CHUNK)], ibuf,
                          isem).wait()

  def _scatter_start(vbuf, ibuf, vsem, csem):
    pltpu.async_copy(vbuf, sums_s.at[ibuf], vsem, add=True)
    pltpu.async_copy(ones_v.at[pl.ds(0, _P1_CHUNK)],
                     cnts_s.at[ibuf], csem, add=True)

  def _scatter_wait(vbuf, ibuf, vsem, csem):
    pltpu.make_async_copy(vbuf, sums_s.at[ibuf], vsem).wait()
    pltpu.make_async_copy(ones_v.at[pl.ds(0, _P1_CHUNK)],
                          cnts_s.at[ibuf], csem).wait()

  _load(0, vals_a, idx_a, sva, sia)
  _load(1, vals_b, idx_b, svb, sib)

  def _fill(i, _):
    zero_v[pl.ds(i * 16, 16)] = jnp.zeros((16,), jnp.float32)
    return 0
  lax.fori_loop(0, _BINS_PER_TILE // 16, _fill, 0)

  def _fill1(i, _):
    ones_v[pl.ds(i * 16, 16)] = jnp.ones((16,), jnp.float32)
    return 0
  lax.fori_loop(0, ones_v.shape[0] // 16, _fill1, 0)

  b0 = pl.multiple_of(sid * _BINS_PER_TILE, 8)
  pltpu.sync_copy(zero_v, sums_s.at[pl.ds(b0, _BINS_PER_TILE)])
  pltpu.sync_copy(zero_v, cnts_s.at[pl.ds(b0, _BINS_PER_TILE)])
  plsc.subcore_barrier()

  def _pipe(g, _):
    _wait_load(vals_a, idx_a, sva, sia)
    _scatter_start(vals_a, idx_a, ssva, ssca)
    _wait_load(vals_b, idx_b, svb, sib)
    _scatter_start(vals_b, idx_b, ssvb, sscb)
    _scatter_wait(vals_a, idx_a, ssva, ssca)

    @pl.when(g < 5)
    def _pfa():
      _load(2 * g + 2, vals_a, idx_a, sva, sia)

    _scatter_wait(vals_b, idx_b, ssvb, sscb)

    @pl.when(g < 5)
    def _pfb():
      _load(2 * g + 3, vals_b, idx_b, svb, sib)

    return 0
  lax.fori_loop(0, 6, _pipe, 0)

  @pl.when(wid < 16)
  def _extra():
    _load(12, vals_a, idx_a, sva, sia)
    _wait_load(vals_a, idx_a, sva, sia)
    _scatter_start(vals_a, idx_a, ssva, ssca)
    _scatter_wait(vals_a, idx_a, ssva, ssca)

  plsc.subcore_barrier()

  po = pl.multiple_of(cid * 2 * _BINS_PAD + b0, 8)
  d1 = pltpu.async_copy(sums_s.at[pl.ds(b0, _BINS_PER_TILE)], zero_v, sva)
  d2 = pltpu.async_copy(cnts_s.at[pl.ds(b0, _BINS_PER_TILE)], stg_v, svb)
  d1.wait()
  d2.wait()
  d3 = pltpu.async_copy(zero_v, acc_hbm.at[pl.ds(po, _BINS_PER_TILE)],
                        sva)
  d4 = pltpu.async_copy(stg_v,
                        acc_hbm.at[pl.ds(po + _BINS_PAD, _BINS_PER_TILE)],
                        svb)
  d3.wait()
  d4.wait()


def _ema_gather_body(acc_hbm, indices_hbm, memory_hbm, out_hbm,
                     newm_s, s0_v, s1_v, c0_v, c1_v, mem_v, newm_v,
                     idx_a, idx_b, out_a, out_b,
                     sia, sib, soa, sob, sld):
  cid = lax.axis_index("c")
  sid = lax.axis_index("s")
  wid = sid * _NC + cid

  b0 = pl.multiple_of(sid * _BINS_PER_TILE, 8)
  p3_base = jnp.minimum(wid * _P3_PER_TILE, _N - _P3_PER_TILE)

  def _cbase(j):
    return pl.multiple_of(p3_base + j * _P3_CHUNK, 8)

  def _load_idx(j, ibuf, isem):
    pltpu.async_copy(indices_hbm.at[pl.ds(_cbase(j), _P3_CHUNK)], ibuf,
                     isem)

  _load_idx(0, idx_a, sia)
  _load_idx(1, idx_b, sib)

  d1 = pltpu.async_copy(acc_hbm.at[pl.ds(b0, _BINS_PER_TILE)], s0_v, sld)
  d2 = pltpu.async_copy(
      acc_hbm.at[pl.ds(2 * _BINS_PAD + b0, _BINS_PER_TILE)], s1_v, sld)
  d3 = pltpu.async_copy(
      acc_hbm.at[pl.ds(_BINS_PAD + b0, _BINS_PER_TILE)], c0_v, sld)
  d4 = pltpu.async_copy(
      acc_hbm.at[pl.ds(3 * _BINS_PAD + b0, _BINS_PER_TILE)], c1_v, sld)
  d5 = pltpu.async_copy(memory_hbm.at[pl.ds(b0, _BINS_PER_TILE)], mem_v,
                        sld)
  d1.wait()
  d2.wait()
  d3.wait()
  d4.wait()
  d5.wait()

  def _p2(k, _):
    o = k * 16
    s = s0_v[pl.ds(o, 16)] + s1_v[pl.ds(o, 16)]
    c = c0_v[pl.ds(o, 16)] + c1_v[pl.ds(o, 16)]
    m = mem_v[pl.ds(o, 16)]
    mean = s / jnp.maximum(c, 1.0)
    alpha = jnp.exp(c * _LN_SMOOTH)
    nm = jnp.where(c > 0.0, alpha * m + (1.0 - alpha) * mean, m)
    newm_v[pl.ds(o, 16)] = nm
    return 0
  lax.fori_loop(0, _BINS_PER_TILE // 16, _p2, 0)

  pltpu.sync_copy(newm_v, newm_s.at[pl.ds(b0, _BINS_PER_TILE)])
  plsc.subcore_barrier()

  def _wait_idx(ibuf, isem):
    pltpu.make_async_copy(indices_hbm.at[pl.ds(0, _P3_CHUNK)], ibuf,
                          isem).wait()

  def _drain_out(obuf, osem):
    pltpu.make_async_copy(obuf, out_hbm.at[pl.ds(0, _P3_CHUNK)],
                          osem).wait()

  def _pipe(g, _):
    _wait_idx(idx_a, sia)

    @pl.when(g > 0)
    def _da():
      _drain_out(out_a, soa)

    pltpu.sync_copy(newm_s.at[idx_a], out_a)
    pltpu.async_copy(out_a, out_hbm.at[pl.ds(_cbase(2 * g), _P3_CHUNK)],
                     soa)
    @pl.when(g < _P3_ITERS // 2 - 1)
    def _pf():
      _load_idx(2 * g + 2, idx_a, sia)

    _wait_idx(idx_b, sib)

    @pl.when(g > 0)
    def _db():
      _drain_out(out_b, sob)

    pltpu.sync_copy(newm_s.at[idx_b], out_b)
    pltpu.async_copy(out_b,
                     out_hbm.at[pl.ds(_cbase(2 * g + 1), _P3_CHUNK)], sob)

    @pl.when(g < _P3_ITERS // 2 - 1)
    def _pfb():
      _load_idx(2 * g + 3, idx_b, sib)

    return 0
  lax.fori_loop(0, _P3_ITERS // 2, _pipe, 0)

  _drain_out(out_a, soa)
  _drain_out(out_b, sob)


@jax.jit
def _smoother(values, indices, memory_padded):
  mesh = plsc.VectorSubcoreMesh(core_axis_name="c", subcore_axis_name="s")
  acc = pl.kernel(
      _acc_body,
      out_type=jax.ShapeDtypeStruct((_NC * 2 * _BINS_PAD,), jnp.float32),
      mesh=mesh,
      scratch_types=[
          pltpu.VMEM_SHARED((_BINS_PAD,), jnp.float32),
          pltpu.VMEM_SHARED((_BINS_PAD,), jnp.float32),
          pltpu.VMEM((_P1_CHUNK,), jnp.float32),
          pltpu.VMEM((_P1_CHUNK,), jnp.float32),
          pltpu.VMEM((_P1_CHUNK,), jnp.int32),
          pltpu.VMEM((_P1_CHUNK,), jnp.int32),
          pltpu.VMEM((5008,), jnp.float32),
          pltpu.VMEM((_BINS_PER_TILE,), jnp.float32),
          pltpu.VMEM((_BINS_PER_TILE,), jnp.float32),
          pltpu.SemaphoreType.DMA,
          pltpu.SemaphoreType.DMA,
          pltpu.SemaphoreType.DMA,
          pltpu.SemaphoreType.DMA,
          pltpu.SemaphoreType.DMA,
          pltpu.SemaphoreType.DMA,
          pltpu.SemaphoreType.DMA,
          pltpu.SemaphoreType.DMA,
      ],
  )(values, indices)

  return pl.kernel(
      _ema_gather_body,
      out_type=jax.ShapeDtypeStruct((_N,), jnp.float32),
      mesh=mesh,
      scratch_types=[
          pltpu.VMEM_SHARED((_BINS_PAD,), jnp.float32),
          pltpu.VMEM((_BINS_PER_TILE,), jnp.float32),
          pltpu.VMEM((_BINS_PER_TILE,), jnp.float32),
          pltpu.VMEM((_BINS_PER_TILE,), jnp.float32),
          pltpu.VMEM((_BINS_PER_TILE,), jnp.float32),
          pltpu.VMEM((_BINS_PER_TILE,), jnp.float32),
          pltpu.VMEM((_BINS_PER_TILE,), jnp.float32),
          pltpu.VMEM((_P3_CHUNK,), jnp.int32),
          pltpu.VMEM((_P3_CHUNK,), jnp.int32),
          pltpu.VMEM((_P3_CHUNK,), jnp.float32),
          pltpu.VMEM((_P3_CHUNK,), jnp.float32),
          pltpu.SemaphoreType.DMA,
          pltpu.SemaphoreType.DMA,
          pltpu.SemaphoreType.DMA,
          pltpu.SemaphoreType.DMA,
          pltpu.SemaphoreType.DMA,
      ],
  )(acc, indices, memory_padded)


def kernel(values, indices, memory):
  memory_padded = jnp.concatenate(
      [memory, jnp.zeros((_BINS_PAD - _NSAMP,), jnp.float32)])
  return _smoother(values, indices, memory_padded)
